# Initial kernel scaffold; baseline (speedup 1.0000x reference)
import jax
import jax.numpy as jnp
from jax.experimental import pallas as pl


def kernel(x, edge_index, edge_attr, edge_type_weights, self_loop_weight, bias):
    raise NotImplementedError("write your pallas kernel here")


# speedup vs baseline: 1.5264x; 1.5264x over previous
import functools

import jax
import jax.numpy as jnp
from jax import lax
from jax.experimental import pallas as pl
from jax.experimental.pallas import tpu as pltpu
from jax.experimental.pallas import tpu_sc as plsc

N_NODES = 10000
N_EDGES = 320000
D_IN = 128
D_OUT = 128
T = 4

NC, NS = 2, 16
NW = NC * NS
EPW = N_EDGES // NW
K = 40
CHUNKS = EPW // K
NPAD = 10240
RPT = NPAD // NS
BF = 2000
BD = 2048


def _matmul_body(x_ref, wcat_ref, wself_ref, y_ref, s_ref):
    xb = x_ref[...]
    y_ref[...] = jnp.dot(xb, wcat_ref[...], preferred_element_type=jnp.float32)
    s_ref[...] = jnp.dot(xb, wself_ref[...], preferred_element_type=jnp.float32)


def _edge_body(y_hbm, src_hbm, tgt_hbm, attr_hbm, parts_hbm, degparts_hbm,
               srcv, tgtv, attrv, rowsv, msgv, degv, aggsh, sem):
    cid = lax.axis_index("c")
    sid = lax.axis_index("s")
    wid = cid * NS + sid

    zero16 = jnp.zeros((16,), jnp.float32)
    one16 = jnp.ones((16,), jnp.float32)
    lane0 = lax.iota(jnp.int32, 16) == 0

    def zmsg(r, _):
        for g in range(D_OUT // 16):
            msgv[r, pl.ds(g * 16, 16)] = zero16
        return 0
    lax.fori_loop(0, K, zmsg, 0)

    def zdeg(r, _):
        degv[pl.ds(r * 16, 16)] = zero16
        return 0
    lax.fori_loop(0, NPAD // 16, zdeg, 0)

    for j in range(RPT // K):
        pltpu.sync_copy(msgv, aggsh.at[pl.ds(sid * RPT + j * K, K)])

    plsc.subcore_barrier()

    ebase = wid * EPW

    def chunk(i, _):
        base = ebase + i * K
        pltpu.sync_copy(src_hbm.at[pl.ds(base, K)], srcv)
        pltpu.sync_copy(tgt_hbm.at[pl.ds(base, K)], tgtv)
        pltpu.sync_copy(attr_hbm.at[pl.ds(base * T, K * T)], attrv)
        pltpu.async_copy(y_hbm.at[srcv], rowsv, sem).wait()

        def edge(e, _):
            eT = e * T
            a0 = plsc.load_gather(attrv, [jnp.full((16,), eT, jnp.int32)])
            a1 = plsc.load_gather(attrv, [jnp.full((16,), eT + 1, jnp.int32)])
            a2 = plsc.load_gather(attrv, [jnp.full((16,), eT + 2, jnp.int32)])
            a3 = plsc.load_gather(attrv, [jnp.full((16,), eT + 3, jnp.int32)])
            for g in range(D_OUT // 16):
                c = g * 16
                acc = a0 * rowsv[e, pl.ds(c, 16)]
                acc = acc + a1 * rowsv[e, pl.ds(D_OUT + c, 16)]
                acc = acc + a2 * rowsv[e, pl.ds(2 * D_OUT + c, 16)]
                acc = acc + a3 * rowsv[e, pl.ds(3 * D_OUT + c, 16)]
                msgv[e, pl.ds(c, 16)] = acc
            tv = plsc.load_gather(tgtv, [jnp.full((16,), e, jnp.int32)])
            plsc.addupdate_scatter(degv, [tv], one16, mask=lane0)
            return 0
        lax.fori_loop(0, K, edge, 0)

        pltpu.sync_copy(msgv, aggsh.at[tgtv], add=True)
        return 0
    lax.fori_loop(0, CHUNKS, chunk, 0)

    pltpu.sync_copy(degv, degparts_hbm.at[wid])

    plsc.subcore_barrier()

    pltpu.sync_copy(aggsh.at[pl.ds(sid * RPT, RPT)],
                    parts_hbm.at[cid, pl.ds(sid * RPT, RPT)])


def _deg_body(dp_ref, r_ref):
    deg = jnp.sum(dp_ref[...], axis=0)
    r_ref[...] = (1.0 / jnp.maximum(deg, 1.0))[:, None]


def _final_body(p_ref, r_ref, s_ref, b_ref, o_ref):
    p = p_ref[...]
    agg = p[0] + p[1]
    o_ref[...] = agg * r_ref[...] + s_ref[...] + b_ref[...]


def kernel(x, edge_index, edge_attr, edge_type_weights, self_loop_weight, bias):
    w_cat = jnp.transpose(edge_type_weights, (1, 0, 2)).reshape(D_IN, T * D_OUT)

    BM = 2000
    y, s = pl.pallas_call(
        _matmul_body,
        grid=(N_NODES // BM,),
        in_specs=[
            pl.BlockSpec((BM, D_IN), lambda i: (i, 0)),
            pl.BlockSpec((D_IN, T * D_OUT), lambda i: (0, 0)),
            pl.BlockSpec((D_IN, D_OUT), lambda i: (0, 0)),
        ],
        out_specs=[
            pl.BlockSpec((BM, T * D_OUT), lambda i: (i, 0)),
            pl.BlockSpec((BM, D_OUT), lambda i: (i, 0)),
        ],
        out_shape=[
            jax.ShapeDtypeStruct((N_NODES, T * D_OUT), jnp.float32),
            jax.ShapeDtypeStruct((N_NODES, D_OUT), jnp.float32),
        ],
    )(x, w_cat, self_loop_weight)

    tgt_hbm = edge_index[0]
    src_hbm = edge_index[1]
    attr_flat = edge_attr.reshape(-1)

    edge_fn = pl.kernel(
        _edge_body,
        out_type=(
            jax.ShapeDtypeStruct((NC, NPAD, D_OUT), jnp.float32),
            jax.ShapeDtypeStruct((NW, NPAD), jnp.float32),
        ),
        mesh=plsc.VectorSubcoreMesh(core_axis_name="c", subcore_axis_name="s"),
        compiler_params=pltpu.CompilerParams(needs_layout_passes=False),
        scratch_types=[
            pltpu.VMEM((K,), jnp.int32),
            pltpu.VMEM((K,), jnp.int32),
            pltpu.VMEM((K * T,), jnp.float32),
            pltpu.VMEM((K, T * D_OUT), jnp.float32),
            pltpu.VMEM((K, D_OUT), jnp.float32),
            pltpu.VMEM((NPAD,), jnp.float32),
            pltpu.VMEM_SHARED((NPAD, D_OUT), jnp.float32),
            pltpu.SemaphoreType.DMA,
        ],
    )
    parts, degparts = edge_fn(y, src_hbm, tgt_hbm, attr_flat)

    recip = pl.pallas_call(
        _deg_body,
        grid=(NPAD // BD,),
        in_specs=[pl.BlockSpec((NW, BD), lambda i: (0, i))],
        out_specs=pl.BlockSpec((BD, 1), lambda i: (i, 0)),
        out_shape=jax.ShapeDtypeStruct((NPAD, 1), jnp.float32),
    )(degparts)

    out = pl.pallas_call(
        _final_body,
        grid=(N_NODES // BF,),
        in_specs=[
            pl.BlockSpec((NC, BF, D_OUT), lambda i: (0, i, 0)),
            pl.BlockSpec((BF, 1), lambda i: (i, 0)),
            pl.BlockSpec((BF, D_OUT), lambda i: (i, 0)),
            pl.BlockSpec((1, D_OUT), lambda i: (0, 0)),
        ],
        out_specs=pl.BlockSpec((BF, D_OUT), lambda i: (i, 0)),
        out_shape=jax.ShapeDtypeStruct((N_NODES, D_OUT), jnp.float32),
    )(parts, recip, s, bias.reshape(1, D_OUT))
    return out


# speedup vs baseline: 2.0447x; 1.3396x over previous
import functools

import numpy as np
import jax
import jax.numpy as jnp
from jax import lax
from jax.experimental import pallas as pl
from jax.experimental.pallas import tpu as pltpu
from jax.experimental.pallas import tpu_sc as plsc

N_NODES = 10000
N_EDGES = 320000
D_IN = 128
D_OUT = 128
T = 4

NC, NS = 2, 16
NW = NC * NS
EPW = N_EDGES // NW
S = 40
CHUNKS = EPW // S
NPAD = 10240
RPT = NPAD // NS
BF = 2000
BD = 2048

_P = np.empty((T * D_OUT,), np.int32)
for _blk in range(T * D_OUT // 32):
    _c = _blk * 32
    for _i in range(16):
        _P[_c + 2 * _i] = _c + _i
        _P[_c + 2 * _i + 1] = _c + 16 + _i


def _matmul_body(x_ref, wcat_ref, wself_ref, y_ref, s_ref):
    xb = x_ref[...]
    y_ref[...] = jnp.dot(
        xb, wcat_ref[...], preferred_element_type=jnp.float32
    ).astype(jnp.bfloat16)
    s_ref[...] = jnp.dot(xb, wself_ref[...], preferred_element_type=jnp.float32)


def _edge_body(y_hbm, src_hbm, tgt_hbm, attr_hbm, parts_hbm, degparts_hbm,
               srcv0, srcv1, tgtv0, tgtv1, attrv0, attrv1,
               rows0, rows1, msg0, msg1, degv, aggsh,
               gsem0, gsem1, ssem0, ssem1):
    cid = lax.axis_index("c")
    sid = lax.axis_index("s")
    wid = cid * NS + sid

    srcv = (srcv0, srcv1)
    tgtv = (tgtv0, tgtv1)
    attrv = (attrv0, attrv1)
    rows = (rows0, rows1)
    msg = (msg0, msg1)
    gsem = (gsem0, gsem1)
    ssem = (ssem0, ssem1)

    zero16 = jnp.zeros((16,), jnp.float32)
    one16 = jnp.ones((16,), jnp.float32)
    lane0 = lax.iota(jnp.int32, 16) == 0

    def zmsg(r, _):
        for g in range(D_OUT // 16):
            msg0[r, pl.ds(g * 16, 16)] = zero16
        return 0
    lax.fori_loop(0, S, zmsg, 0)

    def zdeg(r, _):
        degv[pl.ds(r * 16, 16)] = zero16
        return 0
    lax.fori_loop(0, NPAD // 16, zdeg, 0)

    for j in range(RPT // S):
        pltpu.sync_copy(msg0, aggsh.at[pl.ds(sid * RPT + j * S, S)])

    plsc.subcore_barrier()

    ebase = wid * EPW

    def prefetch(j, b):
        base = ebase + j * S
        pltpu.sync_copy(src_hbm.at[pl.ds(base, S)], srcv[b])
        pltpu.sync_copy(tgt_hbm.at[pl.ds(base, S)], tgtv[b])
        pltpu.sync_copy(attr_hbm.at[pl.ds(base * T, S * T)], attrv[b])
        pltpu.async_copy(y_hbm.at[srcv[b]], rows[b], gsem[b])

    prefetch(0, 0)

    def outer(i, _):
        for b in range(2):
            j = i * 2 + b
            nb = b ^ 1
            rv, mv, av, tv = rows[b], msg[b], attrv[b], tgtv[b]
            pltpu.make_async_copy(y_hbm.at[srcv[b]], rv, gsem[b]).wait()

            @pl.when(j >= 1)
            def _():
                pltpu.make_async_copy(
                    msg[nb], aggsh.at[tgtv[nb]], ssem[nb]).wait()

            @pl.when(j < CHUNKS - 1)
            def _():
                prefetch(j + 1, nb)

            def edge(e, _):
                eT = e * T
                a0 = plsc.load_gather(av, [jnp.full((16,), eT, jnp.int32)])
                a1 = plsc.load_gather(av, [jnp.full((16,), eT + 1, jnp.int32)])
                a2 = plsc.load_gather(av, [jnp.full((16,), eT + 2, jnp.int32)])
                a3 = plsc.load_gather(av, [jnp.full((16,), eT + 3, jnp.int32)])
                p0 = plsc.pack(a0, a0, format=plsc.PackFormat.INTERLEAVED)
                p1 = plsc.pack(a1, a1, format=plsc.PackFormat.INTERLEAVED)
                p2 = plsc.pack(a2, a2, format=plsc.PackFormat.INTERLEAVED)
                p3 = plsc.pack(a3, a3, format=plsc.PackFormat.INTERLEAVED)
                for cb in range(D_OUT // 32):
                    c = cb * 16
                    acc = p0 * plsc.bitcast(rv[e, pl.ds(c, 16)], jnp.bfloat16)
                    acc = acc + p1 * plsc.bitcast(rv[e, pl.ds(64 + c, 16)], jnp.bfloat16)
                    acc = acc + p2 * plsc.bitcast(rv[e, pl.ds(128 + c, 16)], jnp.bfloat16)
                    acc = acc + p3 * plsc.bitcast(rv[e, pl.ds(192 + c, 16)], jnp.bfloat16)
                    c = cb * 32
                    lo, hi = plsc.unpack(acc, format=plsc.PackFormat.INTERLEAVED)
                    mv[e, pl.ds(c, 16)] = lo
                    mv[e, pl.ds(c + 16, 16)] = hi
                t16 = plsc.load_gather(tv, [jnp.full((16,), e, jnp.int32)])
                plsc.addupdate_scatter(degv, [t16], one16, mask=lane0)
                return 0
            lax.fori_loop(0, S, edge, 0)

            pltpu.async_copy(mv, aggsh.at[tv], ssem[b], add=True)
        return 0
    lax.fori_loop(0, CHUNKS // 2, outer, 0)

    pltpu.make_async_copy(msg[1], aggsh.at[tgtv[1]], ssem[1]).wait()

    pltpu.sync_copy(degv, degparts_hbm.at[wid])

    plsc.subcore_barrier()

    pltpu.sync_copy(aggsh.at[pl.ds(sid * RPT, RPT)],
                    parts_hbm.at[cid, pl.ds(sid * RPT, RPT)])


def _deg_body(dp_ref, r_ref):
    deg = jnp.sum(dp_ref[...], axis=0)
    r_ref[...] = (1.0 / jnp.maximum(deg, 1.0))[:, None]


def _final_body(p_ref, r_ref, s_ref, b_ref, o_ref):
    p = p_ref[...]
    agg = p[0] + p[1]
    o_ref[...] = agg * r_ref[...] + s_ref[...] + b_ref[...]


def kernel(x, edge_index, edge_attr, edge_type_weights, self_loop_weight, bias):
    w_cat = jnp.transpose(edge_type_weights, (1, 0, 2)).reshape(D_IN, T * D_OUT)
    w_cat = w_cat[:, _P]

    BM = 2000
    y, s = pl.pallas_call(
        _matmul_body,
        grid=(N_NODES // BM,),
        in_specs=[
            pl.BlockSpec((BM, D_IN), lambda i: (i, 0)),
            pl.BlockSpec((D_IN, T * D_OUT), lambda i: (0, 0)),
            pl.BlockSpec((D_IN, D_OUT), lambda i: (0, 0)),
        ],
        out_specs=[
            pl.BlockSpec((BM, T * D_OUT), lambda i: (i, 0)),
            pl.BlockSpec((BM, D_OUT), lambda i: (i, 0)),
        ],
        out_shape=[
            jax.ShapeDtypeStruct((N_NODES, T * D_OUT), jnp.bfloat16),
            jax.ShapeDtypeStruct((N_NODES, D_OUT), jnp.float32),
        ],
    )(x, w_cat, self_loop_weight)

    y3 = lax.bitcast_convert_type(
        y.reshape(N_NODES, T * D_OUT // 2, 2), jnp.int32
    )
    tgt_hbm = edge_index[0]
    src_hbm = edge_index[1]
    attr_flat = edge_attr.reshape(-1)

    edge_fn = pl.kernel(
        _edge_body,
        out_type=(
            jax.ShapeDtypeStruct((NC, NPAD, D_OUT), jnp.float32),
            jax.ShapeDtypeStruct((NW, NPAD), jnp.float32),
        ),
        mesh=plsc.VectorSubcoreMesh(core_axis_name="c", subcore_axis_name="s"),
        compiler_params=pltpu.CompilerParams(needs_layout_passes=False),
        scratch_types=[
            pltpu.VMEM((S,), jnp.int32),
            pltpu.VMEM((S,), jnp.int32),
            pltpu.VMEM((S,), jnp.int32),
            pltpu.VMEM((S,), jnp.int32),
            pltpu.VMEM((S * T,), jnp.float32),
            pltpu.VMEM((S * T,), jnp.float32),
            pltpu.VMEM((S, T * D_OUT // 2), jnp.int32),
            pltpu.VMEM((S, T * D_OUT // 2), jnp.int32),
            pltpu.VMEM((S, D_OUT), jnp.float32),
            pltpu.VMEM((S, D_OUT), jnp.float32),
            pltpu.VMEM((NPAD,), jnp.float32),
            pltpu.VMEM_SHARED((NPAD, D_OUT), jnp.float32),
            pltpu.SemaphoreType.DMA,
            pltpu.SemaphoreType.DMA,
            pltpu.SemaphoreType.DMA,
            pltpu.SemaphoreType.DMA,
        ],
    )
    parts, degparts = edge_fn(y3, src_hbm, tgt_hbm, attr_flat)

    recip = pl.pallas_call(
        _deg_body,
        grid=(NPAD // BD,),
        in_specs=[pl.BlockSpec((NW, BD), lambda i: (0, i))],
        out_specs=pl.BlockSpec((BD, 1), lambda i: (i, 0)),
        out_shape=jax.ShapeDtypeStruct((NPAD, 1), jnp.float32),
    )(degparts)

    out = pl.pallas_call(
        _final_body,
        grid=(N_NODES // BF,),
        in_specs=[
            pl.BlockSpec((NC, BF, D_OUT), lambda i: (0, i, 0)),
            pl.BlockSpec((BF, 1), lambda i: (i, 0)),
            pl.BlockSpec((BF, D_OUT), lambda i: (i, 0)),
            pl.BlockSpec((1, D_OUT), lambda i: (0, 0)),
        ],
        out_specs=pl.BlockSpec((BF, D_OUT), lambda i: (i, 0)),
        out_shape=jax.ShapeDtypeStruct((N_NODES, D_OUT), jnp.float32),
    )(parts, recip, s, bias.reshape(1, D_OUT))
    return out


# speedup vs baseline: 3.5646x; 1.7433x over previous
import functools

import numpy as np
import jax
import jax.numpy as jnp
from jax import lax
from jax.experimental import pallas as pl
from jax.experimental.pallas import tpu as pltpu
from jax.experimental.pallas import tpu_sc as plsc

N_NODES = 10000
N_EDGES = 320000
D_IN = 128
D_OUT = 128
T = 4

NC, NS = 2, 16
NW = NC * NS
EPW = N_EDGES // NW
S = 40
CHUNKS = EPW // S
NPAD = 10240
RPT = NPAD // NS
BF = 2000
BD = 2048

_P = np.empty((T * D_OUT,), np.int32)
for _blk in range(T * D_OUT // 32):
    _c = _blk * 32
    for _i in range(16):
        _P[_c + 2 * _i] = _c + _i
        _P[_c + 2 * _i + 1] = _c + 16 + _i


def _matmul_body(x_ref, wcat_ref, wself_ref, y_ref, s_ref):
    xb = x_ref[...]
    y_ref[...] = jnp.dot(
        xb, wcat_ref[...], preferred_element_type=jnp.float32
    ).astype(jnp.bfloat16)
    s_ref[...] = jnp.dot(xb, wself_ref[...], preferred_element_type=jnp.float32)


def _edge_body(y_hbm, src_hbm, tgt_hbm, attr_hbm, parts_hbm, degparts_hbm,
               srcv0, srcv1, tgtv0, tgtv1, attrv0, attrv1, tsc0, tsc1,
               rows0, rows1, msg0, msg1, degv, aggsh,
               gsem0, gsem1, ssem0, ssem1, isem0, isem1):
    cid = lax.axis_index("c")
    sid = lax.axis_index("s")
    wid = cid * NS + sid

    srcv = (srcv0, srcv1)
    tgtv = (tgtv0, tgtv1)
    attrv = (attrv0, attrv1)
    tsc = (tsc0, tsc1)
    rows = (rows0, rows1)
    msg = (msg0, msg1)
    gsem = (gsem0, gsem1)
    ssem = (ssem0, ssem1)
    isem = (isem0, isem1)

    zero16 = jnp.zeros((16,), jnp.float32)
    one16 = jnp.ones((16,), jnp.float32)
    lane0 = lax.iota(jnp.int32, 16) == 0

    def zmsg(r, _):
        for g in range(D_OUT // 16):
            msg0[r, pl.ds(g * 16, 16)] = zero16
        return 0
    lax.fori_loop(0, S, zmsg, 0)

    def zdeg(r, _):
        degv[pl.ds(r * 16, 16)] = zero16
        return 0
    lax.fori_loop(0, NPAD // 16, zdeg, 0)

    for j in range(RPT // S):
        pltpu.sync_copy(msg0, aggsh.at[pl.ds(sid * RPT + j * S, S)])

    plsc.subcore_barrier()

    ebase = wid * EPW

    def issue_idx(j, b):
        base = ebase + j * S
        pltpu.async_copy(src_hbm.at[pl.ds(base, S)], srcv[b], isem[b])
        pltpu.async_copy(tgt_hbm.at[pl.ds(base, S)], tgtv[b], isem[b])
        pltpu.async_copy(attr_hbm.at[pl.ds(base * T, S * T)], attrv[b], isem[b])

    def wait_idx(j, b):
        base = ebase + j * S
        pltpu.make_async_copy(src_hbm.at[pl.ds(base, S)], srcv[b], isem[b]).wait()
        pltpu.make_async_copy(tgt_hbm.at[pl.ds(base, S)], tgtv[b], isem[b]).wait()
        pltpu.make_async_copy(attr_hbm.at[pl.ds(base * T, S * T)], attrv[b], isem[b]).wait()

    issue_idx(0, 0)
    issue_idx(1, 1)
    wait_idx(0, 0)
    pltpu.async_copy(y_hbm.at[srcv[0]], rows[0], gsem[0])

    def outer(i, _):
        for b in range(2):
            j = i * 2 + b
            nb = b ^ 1
            rv, mv, av = rows[b], msg[b], attrv[b]
            pltpu.make_async_copy(y_hbm.at[srcv[b]], rv, gsem[b]).wait()

            @pl.when(j >= 1)
            def _():
                pltpu.make_async_copy(
                    msg[nb], aggsh.at[tsc[nb]], ssem[nb]).wait()

            for q in (0, 16, S - 16):
                tsc[b][pl.ds(q, 16)] = tgtv[b][pl.ds(q, 16)]

            @pl.when(j + 1 < CHUNKS)
            def _():
                wait_idx(j + 1, nb)
                pltpu.async_copy(y_hbm.at[srcv[nb]], rows[nb], gsem[nb])

            @plsc.parallel_loop(0, S, unroll=2)
            def _(e):
                eT = e * T
                a0 = plsc.load_gather(av, [jnp.full((16,), eT, jnp.int32)])
                a1 = plsc.load_gather(av, [jnp.full((16,), eT + 1, jnp.int32)])
                a2 = plsc.load_gather(av, [jnp.full((16,), eT + 2, jnp.int32)])
                a3 = plsc.load_gather(av, [jnp.full((16,), eT + 3, jnp.int32)])
                p0 = plsc.pack(a0, a0, format=plsc.PackFormat.INTERLEAVED)
                p1 = plsc.pack(a1, a1, format=plsc.PackFormat.INTERLEAVED)
                p2 = plsc.pack(a2, a2, format=plsc.PackFormat.INTERLEAVED)
                p3 = plsc.pack(a3, a3, format=plsc.PackFormat.INTERLEAVED)
                for cb in range(D_OUT // 32):
                    c = cb * 16
                    acc = p0 * plsc.bitcast(rv[e, pl.ds(c, 16)], jnp.bfloat16)
                    acc = acc + p1 * plsc.bitcast(rv[e, pl.ds(64 + c, 16)], jnp.bfloat16)
                    acc = acc + p2 * plsc.bitcast(rv[e, pl.ds(128 + c, 16)], jnp.bfloat16)
                    acc = acc + p3 * plsc.bitcast(rv[e, pl.ds(192 + c, 16)], jnp.bfloat16)
                    c = cb * 32
                    lo, hi = plsc.unpack(acc, format=plsc.PackFormat.INTERLEAVED)
                    mv[e, pl.ds(c, 16)] = lo
                    mv[e, pl.ds(c + 16, 16)] = hi
                t16 = plsc.load_gather(tsc[b], [jnp.full((16,), e, jnp.int32)])
                plsc.addupdate_scatter(degv, [t16], one16, mask=lane0)

            pltpu.async_copy(mv, aggsh.at[tsc[b]], ssem[b], add=True)

            @pl.when(j + 2 < CHUNKS)
            def _():
                issue_idx(j + 2, b)
        return 0
    lax.fori_loop(0, CHUNKS // 2, outer, 0)

    pltpu.make_async_copy(msg[1], aggsh.at[tsc[1]], ssem[1]).wait()

    pltpu.sync_copy(degv, degparts_hbm.at[wid])

    plsc.subcore_barrier()

    pltpu.sync_copy(aggsh.at[pl.ds(sid * RPT, RPT)],
                    parts_hbm.at[cid, pl.ds(sid * RPT, RPT)])


def _deg_body(dp_ref, r_ref):
    deg = jnp.sum(dp_ref[...], axis=0)
    r_ref[...] = (1.0 / jnp.maximum(deg, 1.0))[:, None]


def _final_body(p_ref, r_ref, s_ref, b_ref, o_ref):
    p = p_ref[...]
    agg = p[0] + p[1]
    o_ref[...] = agg * r_ref[...] + s_ref[...] + b_ref[...]


def kernel(x, edge_index, edge_attr, edge_type_weights, self_loop_weight, bias):
    w_cat = jnp.transpose(edge_type_weights, (1, 0, 2)).reshape(D_IN, T * D_OUT)
    w_cat = w_cat[:, _P]

    BM = 2000
    y, s = pl.pallas_call(
        _matmul_body,
        grid=(N_NODES // BM,),
        in_specs=[
            pl.BlockSpec((BM, D_IN), lambda i: (i, 0)),
            pl.BlockSpec((D_IN, T * D_OUT), lambda i: (0, 0)),
            pl.BlockSpec((D_IN, D_OUT), lambda i: (0, 0)),
        ],
        out_specs=[
            pl.BlockSpec((BM, T * D_OUT), lambda i: (i, 0)),
            pl.BlockSpec((BM, D_OUT), lambda i: (i, 0)),
        ],
        out_shape=[
            jax.ShapeDtypeStruct((N_NODES, T * D_OUT), jnp.bfloat16),
            jax.ShapeDtypeStruct((N_NODES, D_OUT), jnp.float32),
        ],
    )(x, w_cat, self_loop_weight)

    y3 = lax.bitcast_convert_type(
        y.reshape(N_NODES, T * D_OUT // 2, 2), jnp.int32
    )
    tgt_hbm = edge_index[0]
    src_hbm = edge_index[1]
    attr_flat = edge_attr.reshape(-1)

    edge_fn = pl.kernel(
        _edge_body,
        out_type=(
            jax.ShapeDtypeStruct((NC, NPAD, D_OUT), jnp.float32),
            jax.ShapeDtypeStruct((NW, NPAD), jnp.float32),
        ),
        mesh=plsc.VectorSubcoreMesh(core_axis_name="c", subcore_axis_name="s"),
        compiler_params=pltpu.CompilerParams(needs_layout_passes=False),
        scratch_types=[
            pltpu.VMEM((S,), jnp.int32),
            pltpu.VMEM((S,), jnp.int32),
            pltpu.VMEM((S,), jnp.int32),
            pltpu.VMEM((S,), jnp.int32),
            pltpu.VMEM((S * T,), jnp.float32),
            pltpu.VMEM((S * T,), jnp.float32),
            pltpu.VMEM((S,), jnp.int32),
            pltpu.VMEM((S,), jnp.int32),
            pltpu.VMEM((S, T * D_OUT // 2), jnp.int32),
            pltpu.VMEM((S, T * D_OUT // 2), jnp.int32),
            pltpu.VMEM((S, D_OUT), jnp.float32),
            pltpu.VMEM((S, D_OUT), jnp.float32),
            pltpu.VMEM((NPAD,), jnp.float32),
            pltpu.VMEM_SHARED((NPAD, D_OUT), jnp.float32),
            pltpu.SemaphoreType.DMA,
            pltpu.SemaphoreType.DMA,
            pltpu.SemaphoreType.DMA,
            pltpu.SemaphoreType.DMA,
            pltpu.SemaphoreType.DMA,
            pltpu.SemaphoreType.DMA,
        ],
    )
    parts, degparts = edge_fn(y3, src_hbm, tgt_hbm, attr_flat)

    recip = pl.pallas_call(
        _deg_body,
        grid=(NPAD // BD,),
        in_specs=[pl.BlockSpec((NW, BD), lambda i: (0, i))],
        out_specs=pl.BlockSpec((BD, 1), lambda i: (i, 0)),
        out_shape=jax.ShapeDtypeStruct((NPAD, 1), jnp.float32),
    )(degparts)

    out = pl.pallas_call(
        _final_body,
        grid=(N_NODES // BF,),
        in_specs=[
            pl.BlockSpec((NC, BF, D_OUT), lambda i: (0, i, 0)),
            pl.BlockSpec((BF, 1), lambda i: (i, 0)),
            pl.BlockSpec((BF, D_OUT), lambda i: (i, 0)),
            pl.BlockSpec((1, D_OUT), lambda i: (0, 0)),
        ],
        out_specs=pl.BlockSpec((BF, D_OUT), lambda i: (i, 0)),
        out_shape=jax.ShapeDtypeStruct((N_NODES, D_OUT), jnp.float32),
    )(parts, recip, s, bias.reshape(1, D_OUT))
    return out


# speedup vs baseline: 4.7371x; 1.3289x over previous
import functools

import numpy as np
import jax
import jax.numpy as jnp
from jax import lax
from jax.experimental import pallas as pl
from jax.experimental.pallas import tpu as pltpu
from jax.experimental.pallas import tpu_sc as plsc

N_NODES = 10000
N_EDGES = 320000
D_IN = 128
D_OUT = 128
T = 4

NC, NS = 2, 16
NW = NC * NS
EPW = N_EDGES // NW
S = 40
CHUNKS = EPW // S
NPAD = 10240
RPT = NPAD // NS
BF = 2000
BD = 2048

_P = np.empty((T * D_OUT,), np.int32)
for _blk in range(T * D_OUT // 32):
    _c = _blk * 32
    for _i in range(16):
        _P[_c + 2 * _i] = _c + _i
        _P[_c + 2 * _i + 1] = _c + 16 + _i


def _matmul_body(x_ref, we_ref, wo_ref, wself_ref, y_ref, s_ref):
    xb = x_ref[...]
    lo = jnp.dot(xb, we_ref[...], preferred_element_type=jnp.float32)
    hi = jnp.dot(xb, wo_ref[...], preferred_element_type=jnp.float32)
    lo16 = lax.bitcast_convert_type(lo.astype(jnp.bfloat16), jnp.uint16)
    hi16 = lax.bitcast_convert_type(hi.astype(jnp.bfloat16), jnp.uint16)
    y_ref[...] = lo16.astype(jnp.int32) | (hi16.astype(jnp.int32) << 16)
    s_ref[...] = jnp.dot(xb, wself_ref[...], preferred_element_type=jnp.float32)


def _edge_body(y_hbm, src_hbm, tgt_hbm, attr_hbm, parts_hbm, degparts_hbm,
               srcv0, srcv1, tgtv0, tgtv1, attrv0, attrv1, tsc0, tsc1,
               rows0, rows1, msg0, msg1, degv, aggsh,
               gsem0, gsem1, ssem0, ssem1, isem0, isem1):
    cid = lax.axis_index("c")
    sid = lax.axis_index("s")
    wid = cid * NS + sid

    srcv = (srcv0, srcv1)
    tgtv = (tgtv0, tgtv1)
    attrv = (attrv0, attrv1)
    tsc = (tsc0, tsc1)
    rows = (rows0, rows1)
    msg = (msg0, msg1)
    gsem = (gsem0, gsem1)
    ssem = (ssem0, ssem1)
    isem = (isem0, isem1)

    zero16 = jnp.zeros((16,), jnp.float32)
    one16 = jnp.ones((16,), jnp.float32)
    lane0 = lax.iota(jnp.int32, 16) == 0

    def zmsg(r, _):
        for g in range(D_OUT // 16):
            msg0[r, pl.ds(g * 16, 16)] = zero16
        return 0
    lax.fori_loop(0, S, zmsg, 0)

    def zdeg(r, _):
        degv[pl.ds(r * 16, 16)] = zero16
        return 0
    lax.fori_loop(0, NPAD // 16, zdeg, 0)

    for j in range(RPT // S):
        pltpu.sync_copy(msg0, aggsh.at[pl.ds(sid * RPT + j * S, S)])

    plsc.subcore_barrier()

    ebase = wid * EPW

    def issue_idx(j, b):
        base = ebase + j * S
        pltpu.async_copy(src_hbm.at[pl.ds(base, S)], srcv[b], isem[b])
        pltpu.async_copy(tgt_hbm.at[pl.ds(base, S)], tgtv[b], isem[b])
        pltpu.async_copy(attr_hbm.at[pl.ds(base * T, S * T)], attrv[b], isem[b])

    def wait_idx(j, b):
        base = ebase + j * S
        pltpu.make_async_copy(src_hbm.at[pl.ds(base, S)], srcv[b], isem[b]).wait()
        pltpu.make_async_copy(tgt_hbm.at[pl.ds(base, S)], tgtv[b], isem[b]).wait()
        pltpu.make_async_copy(attr_hbm.at[pl.ds(base * T, S * T)], attrv[b], isem[b]).wait()

    issue_idx(0, 0)
    issue_idx(1, 1)
    wait_idx(0, 0)
    pltpu.async_copy(y_hbm.at[srcv[0]], rows[0], gsem[0])

    def outer(i, _):
        for b in range(2):
            j = i * 2 + b
            nb = b ^ 1
            rv, mv, av = rows[b], msg[b], attrv[b]
            pltpu.make_async_copy(y_hbm.at[srcv[b]], rv, gsem[b]).wait()

            @pl.when(j >= 1)
            def _():
                pltpu.make_async_copy(
                    msg[nb], aggsh.at[tsc[nb]], ssem[nb]).wait()

            for q in (0, 16, S - 16):
                tsc[b][pl.ds(q, 16)] = tgtv[b][pl.ds(q, 16)]

            @pl.when(j + 1 < CHUNKS)
            def _():
                wait_idx(j + 1, nb)
                pltpu.async_copy(y_hbm.at[srcv[nb]], rows[nb], gsem[nb])

            @plsc.parallel_loop(0, S, unroll=2)
            def _(e):
                eT = e * T
                a0 = plsc.load_gather(av, [jnp.full((16,), eT, jnp.int32)])
                a1 = plsc.load_gather(av, [jnp.full((16,), eT + 1, jnp.int32)])
                a2 = plsc.load_gather(av, [jnp.full((16,), eT + 2, jnp.int32)])
                a3 = plsc.load_gather(av, [jnp.full((16,), eT + 3, jnp.int32)])
                p0 = plsc.pack(a0, a0, format=plsc.PackFormat.INTERLEAVED)
                p1 = plsc.pack(a1, a1, format=plsc.PackFormat.INTERLEAVED)
                p2 = plsc.pack(a2, a2, format=plsc.PackFormat.INTERLEAVED)
                p3 = plsc.pack(a3, a3, format=plsc.PackFormat.INTERLEAVED)
                for cb in range(D_OUT // 32):
                    c = cb * 16
                    acc = p0 * plsc.bitcast(rv[e, pl.ds(c, 16)], jnp.bfloat16)
                    acc = acc + p1 * plsc.bitcast(rv[e, pl.ds(64 + c, 16)], jnp.bfloat16)
                    acc = acc + p2 * plsc.bitcast(rv[e, pl.ds(128 + c, 16)], jnp.bfloat16)
                    acc = acc + p3 * plsc.bitcast(rv[e, pl.ds(192 + c, 16)], jnp.bfloat16)
                    c = cb * 32
                    lo, hi = plsc.unpack(acc, format=plsc.PackFormat.INTERLEAVED)
                    mv[e, pl.ds(c, 16)] = lo
                    mv[e, pl.ds(c + 16, 16)] = hi
                t16 = plsc.load_gather(tsc[b], [jnp.full((16,), e, jnp.int32)])
                plsc.addupdate_scatter(degv, [t16], one16, mask=lane0)

            pltpu.async_copy(mv, aggsh.at[tsc[b]], ssem[b], add=True)

            @pl.when(j + 2 < CHUNKS)
            def _():
                issue_idx(j + 2, b)
        return 0
    lax.fori_loop(0, CHUNKS // 2, outer, 0)

    pltpu.make_async_copy(msg[1], aggsh.at[tsc[1]], ssem[1]).wait()

    pltpu.sync_copy(degv, degparts_hbm.at[wid])

    plsc.subcore_barrier()

    pltpu.sync_copy(aggsh.at[pl.ds(sid * RPT, RPT)],
                    parts_hbm.at[cid, pl.ds(sid * RPT, RPT)])


def _deg_body(dp_ref, r_ref):
    deg = jnp.sum(dp_ref[...], axis=0)
    r_ref[...] = (1.0 / jnp.maximum(deg, 1.0))[:, None]


def _final_body(p_ref, r_ref, s_ref, b_ref, o_ref):
    p = p_ref[...]
    agg = p[0] + p[1]
    o_ref[...] = agg * r_ref[...] + s_ref[...] + b_ref[...]


def kernel(x, edge_index, edge_attr, edge_type_weights, self_loop_weight, bias):
    w_cat = jnp.transpose(edge_type_weights, (1, 0, 2)).reshape(D_IN, T * D_OUT)
    w_even = w_cat[:, _P[0::2]]
    w_odd = w_cat[:, _P[1::2]]

    BM = 2000
    HW = T * D_OUT // 2
    y3, s = pl.pallas_call(
        _matmul_body,
        grid=(N_NODES // BM,),
        in_specs=[
            pl.BlockSpec((BM, D_IN), lambda i: (i, 0)),
            pl.BlockSpec((D_IN, HW), lambda i: (0, 0)),
            pl.BlockSpec((D_IN, HW), lambda i: (0, 0)),
            pl.BlockSpec((D_IN, D_OUT), lambda i: (0, 0)),
        ],
        out_specs=[
            pl.BlockSpec((BM, HW), lambda i: (i, 0)),
            pl.BlockSpec((BM, D_OUT), lambda i: (i, 0)),
        ],
        out_shape=[
            jax.ShapeDtypeStruct((N_NODES, HW), jnp.int32),
            jax.ShapeDtypeStruct((N_NODES, D_OUT), jnp.float32),
        ],
    )(x, w_even, w_odd, self_loop_weight)

    tgt_hbm = edge_index[0]
    src_hbm = edge_index[1]
    attr_flat = edge_attr.reshape(-1)

    edge_fn = pl.kernel(
        _edge_body,
        out_type=(
            jax.ShapeDtypeStruct((NC, NPAD, D_OUT), jnp.float32),
            jax.ShapeDtypeStruct((NW, NPAD), jnp.float32),
        ),
        mesh=plsc.VectorSubcoreMesh(core_axis_name="c", subcore_axis_name="s"),
        compiler_params=pltpu.CompilerParams(needs_layout_passes=False),
        scratch_types=[
            pltpu.VMEM((S,), jnp.int32),
            pltpu.VMEM((S,), jnp.int32),
            pltpu.VMEM((S,), jnp.int32),
            pltpu.VMEM((S,), jnp.int32),
            pltpu.VMEM((S * T,), jnp.float32),
            pltpu.VMEM((S * T,), jnp.float32),
            pltpu.VMEM((S,), jnp.int32),
            pltpu.VMEM((S,), jnp.int32),
            pltpu.VMEM((S, T * D_OUT // 2), jnp.int32),
            pltpu.VMEM((S, T * D_OUT // 2), jnp.int32),
            pltpu.VMEM((S, D_OUT), jnp.float32),
            pltpu.VMEM((S, D_OUT), jnp.float32),
            pltpu.VMEM((NPAD,), jnp.float32),
            pltpu.VMEM_SHARED((NPAD, D_OUT), jnp.float32),
            pltpu.SemaphoreType.DMA,
            pltpu.SemaphoreType.DMA,
            pltpu.SemaphoreType.DMA,
            pltpu.SemaphoreType.DMA,
            pltpu.SemaphoreType.DMA,
            pltpu.SemaphoreType.DMA,
        ],
    )
    parts, degparts = edge_fn(y3, src_hbm, tgt_hbm, attr_flat)

    recip = pl.pallas_call(
        _deg_body,
        grid=(NPAD // BD,),
        in_specs=[pl.BlockSpec((NW, BD), lambda i: (0, i))],
        out_specs=pl.BlockSpec((BD, 1), lambda i: (i, 0)),
        out_shape=jax.ShapeDtypeStruct((NPAD, 1), jnp.float32),
    )(degparts)

    out = pl.pallas_call(
        _final_body,
        grid=(N_NODES // BF,),
        in_specs=[
            pl.BlockSpec((NC, BF, D_OUT), lambda i: (0, i, 0)),
            pl.BlockSpec((BF, 1), lambda i: (i, 0)),
            pl.BlockSpec((BF, D_OUT), lambda i: (i, 0)),
            pl.BlockSpec((1, D_OUT), lambda i: (0, 0)),
        ],
        out_specs=pl.BlockSpec((BF, D_OUT), lambda i: (i, 0)),
        out_shape=jax.ShapeDtypeStruct((N_NODES, D_OUT), jnp.float32),
    )(parts, recip, s, bias.reshape(1, D_OUT))
    return out


# speedup vs baseline: 5.1442x; 1.0859x over previous
import functools

import numpy as np
import jax
import jax.numpy as jnp
from jax import lax
from jax.experimental import pallas as pl
from jax.experimental.pallas import tpu as pltpu
from jax.experimental.pallas import tpu_sc as plsc

N_NODES = 10000
N_EDGES = 320000
D_IN = 128
D_OUT = 128
T = 4

NC, NS = 2, 16
NW = NC * NS
EPW = N_EDGES // NW
S = 40
CHUNKS = EPW // S
NPAD = 10240
RPT = NPAD // NS
BF = 2000
BD = 2048
BE = 3200

_P = np.empty((T * D_OUT,), np.int32)
for _blk in range(T * D_OUT // 32):
    _c = _blk * 32
    for _i in range(16):
        _P[_c + 2 * _i] = _c + _i
        _P[_c + 2 * _i + 1] = _c + 16 + _i


def _matmul_body(x_ref, we_ref, wo_ref, wself_ref, y_ref, s_ref):
    xb = x_ref[...]
    lo = jnp.dot(xb, we_ref[...], preferred_element_type=jnp.float32)
    hi = jnp.dot(xb, wo_ref[...], preferred_element_type=jnp.float32)
    lo16 = lax.bitcast_convert_type(lo.astype(jnp.bfloat16), jnp.uint16)
    hi16 = lax.bitcast_convert_type(hi.astype(jnp.bfloat16), jnp.uint16)
    y_ref[...] = lo16.astype(jnp.int32) | (hi16.astype(jnp.int32) << 16)
    s_ref[...] = jnp.dot(xb, wself_ref[...], preferred_element_type=jnp.float32)


def _edge_body(y_hbm, src_hbm, tgt_hbm, a0_hbm, a1_hbm, a2_hbm, a3_hbm,
               parts_hbm, degparts_hbm,
               srcv0, srcv1, tgtv0, tgtv1, attrv0, attrv1, tsc0, tsc1,
               rows0, rows1, msg0, msg1, degv, aggsh,
               gsem0, gsem1, ssem0, ssem1, isem0, isem1):
    cid = lax.axis_index("c")
    sid = lax.axis_index("s")
    wid = cid * NS + sid

    srcv = (srcv0, srcv1)
    tgtv = (tgtv0, tgtv1)
    attrv = (attrv0, attrv1)
    tsc = (tsc0, tsc1)
    rows = (rows0, rows1)
    msg = (msg0, msg1)
    gsem = (gsem0, gsem1)
    ssem = (ssem0, ssem1)
    isem = (isem0, isem1)

    zero16 = jnp.zeros((16,), jnp.float32)
    one16 = jnp.ones((16,), jnp.float32)
    lane0 = lax.iota(jnp.int32, 16) == 0

    def zmsg(r, _):
        for g in range(D_OUT // 16):
            msg0[r, pl.ds(g * 16, 16)] = zero16
        return 0
    lax.fori_loop(0, S, zmsg, 0)

    def zdeg(r, _):
        degv[pl.ds(r * 16, 16)] = zero16
        return 0
    lax.fori_loop(0, NPAD // 16, zdeg, 0)

    for j in range(RPT // S):
        pltpu.sync_copy(msg0, aggsh.at[pl.ds(sid * RPT + j * S, S)])

    plsc.subcore_barrier()

    ebase = wid * EPW

    ahbm = (a0_hbm, a1_hbm, a2_hbm, a3_hbm)

    def issue_idx(j, b):
        base = ebase + j * S
        pltpu.async_copy(src_hbm.at[pl.ds(base, S)], srcv[b], isem[b])
        pltpu.async_copy(tgt_hbm.at[pl.ds(base, S)], tgtv[b], isem[b])
        for t in range(T):
            pltpu.async_copy(ahbm[t].at[pl.ds(base, S)],
                             attrv[b].at[pl.ds(t * S, S)], isem[b])

    def wait_idx(j, b):
        base = ebase + j * S
        pltpu.make_async_copy(src_hbm.at[pl.ds(base, S)], srcv[b], isem[b]).wait()
        pltpu.make_async_copy(tgt_hbm.at[pl.ds(base, S)], tgtv[b], isem[b]).wait()
        for t in range(T):
            pltpu.make_async_copy(ahbm[t].at[pl.ds(base, S)],
                                  attrv[b].at[pl.ds(t * S, S)], isem[b]).wait()

    issue_idx(0, 0)
    issue_idx(1, 1)
    wait_idx(0, 0)
    pltpu.async_copy(y_hbm.at[srcv[0]], rows[0], gsem[0])

    def outer(i, _):
        for b in range(2):
            j = i * 2 + b
            nb = b ^ 1
            rv, mv, av = rows[b], msg[b], attrv[b]
            pltpu.make_async_copy(y_hbm.at[srcv[b]], rv, gsem[b]).wait()

            @pl.when(j >= 1)
            def _():
                pltpu.make_async_copy(
                    msg[nb], aggsh.at[tsc[nb]], ssem[nb]).wait()

            for q in (0, 16, S - 16):
                tsc[b][pl.ds(q, 16)] = tgtv[b][pl.ds(q, 16)]

            @pl.when(j + 1 < CHUNKS)
            def _():
                wait_idx(j + 1, nb)
                pltpu.async_copy(y_hbm.at[srcv[nb]], rows[nb], gsem[nb])

            @plsc.parallel_loop(0, S, unroll=2)
            def _(e):
                a0 = plsc.load_gather(av, [jnp.full((16,), e, jnp.int32)])
                a1 = plsc.load_gather(av, [jnp.full((16,), S + e, jnp.int32)])
                a2 = plsc.load_gather(av, [jnp.full((16,), 2 * S + e, jnp.int32)])
                a3 = plsc.load_gather(av, [jnp.full((16,), 3 * S + e, jnp.int32)])
                p0 = plsc.pack(a0, a0, format=plsc.PackFormat.INTERLEAVED)
                p1 = plsc.pack(a1, a1, format=plsc.PackFormat.INTERLEAVED)
                p2 = plsc.pack(a2, a2, format=plsc.PackFormat.INTERLEAVED)
                p3 = plsc.pack(a3, a3, format=plsc.PackFormat.INTERLEAVED)
                for cb in range(D_OUT // 32):
                    c = cb * 16
                    acc = p0 * plsc.bitcast(rv[e, pl.ds(c, 16)], jnp.bfloat16)
                    acc = acc + p1 * plsc.bitcast(rv[e, pl.ds(64 + c, 16)], jnp.bfloat16)
                    acc = acc + p2 * plsc.bitcast(rv[e, pl.ds(128 + c, 16)], jnp.bfloat16)
                    acc = acc + p3 * plsc.bitcast(rv[e, pl.ds(192 + c, 16)], jnp.bfloat16)
                    c = cb * 32
                    lo, hi = plsc.unpack(acc, format=plsc.PackFormat.INTERLEAVED)
                    mv[e, pl.ds(c, 16)] = lo
                    mv[e, pl.ds(c + 16, 16)] = hi
                t16 = plsc.load_gather(tsc[b], [jnp.full((16,), e, jnp.int32)])
                plsc.addupdate_scatter(degv, [t16], one16, mask=lane0)

            pltpu.async_copy(mv, aggsh.at[tsc[b]], ssem[b], add=True)

            @pl.when(j + 2 < CHUNKS)
            def _():
                issue_idx(j + 2, b)
        return 0
    lax.fori_loop(0, CHUNKS // 2, outer, 0)

    pltpu.make_async_copy(msg[1], aggsh.at[tsc[1]], ssem[1]).wait()

    pltpu.sync_copy(degv, degparts_hbm.at[wid])

    plsc.subcore_barrier()

    pltpu.sync_copy(aggsh.at[pl.ds(sid * RPT, RPT)],
                    parts_hbm.at[cid, pl.ds(sid * RPT, RPT)])


def _linearize_body(eidx_ref, attr_ref, src_ref, tgt_ref,
                    a0_ref, a1_ref, a2_ref, a3_ref):
    i = pl.program_id(0)
    sl = pl.ds(i * BE, BE)
    ei = eidx_ref[...]
    src_ref[sl] = ei[1]
    tgt_ref[sl] = ei[0]
    tr = jnp.transpose(attr_ref[...])
    a0_ref[sl] = tr[0]
    a1_ref[sl] = tr[1]
    a2_ref[sl] = tr[2]
    a3_ref[sl] = tr[3]


def _deg_body(dp_ref, r_ref):
    deg = jnp.sum(dp_ref[...], axis=0)
    r_ref[...] = (1.0 / jnp.maximum(deg, 1.0))[:, None]


def _final_body(p_ref, r_ref, s_ref, b_ref, o_ref):
    p = p_ref[...]
    agg = p[0] + p[1]
    o_ref[...] = agg * r_ref[...] + s_ref[...] + b_ref[...]


def kernel(x, edge_index, edge_attr, edge_type_weights, self_loop_weight, bias):
    w_cat = jnp.transpose(edge_type_weights, (1, 0, 2)).reshape(D_IN, T * D_OUT)
    w_even = w_cat[:, _P[0::2]]
    w_odd = w_cat[:, _P[1::2]]

    BM = 2000
    HW = T * D_OUT // 2
    y3, s = pl.pallas_call(
        _matmul_body,
        grid=(N_NODES // BM,),
        in_specs=[
            pl.BlockSpec((BM, D_IN), lambda i: (i, 0)),
            pl.BlockSpec((D_IN, HW), lambda i: (0, 0)),
            pl.BlockSpec((D_IN, HW), lambda i: (0, 0)),
            pl.BlockSpec((D_IN, D_OUT), lambda i: (0, 0)),
        ],
        out_specs=[
            pl.BlockSpec((BM, HW), lambda i: (i, 0)),
            pl.BlockSpec((BM, D_OUT), lambda i: (i, 0)),
        ],
        out_shape=[
            jax.ShapeDtypeStruct((N_NODES, HW), jnp.int32),
            jax.ShapeDtypeStruct((N_NODES, D_OUT), jnp.float32),
        ],
    )(x, w_even, w_odd, self_loop_weight)

    src_l, tgt_l, a0_l, a1_l, a2_l, a3_l = pl.pallas_call(
        _linearize_body,
        grid=(N_EDGES // BE,),
        in_specs=[
            pl.BlockSpec((2, BE), lambda i: (0, i)),
            pl.BlockSpec((BE, T), lambda i: (i, 0)),
        ],
        out_specs=[pl.BlockSpec((N_EDGES,), lambda i: (0,))] * 6,
        out_shape=[
            jax.ShapeDtypeStruct((N_EDGES,), jnp.int32),
            jax.ShapeDtypeStruct((N_EDGES,), jnp.int32),
        ] + [jax.ShapeDtypeStruct((N_EDGES,), jnp.float32)] * T,
    )(edge_index, edge_attr)

    edge_fn = pl.kernel(
        _edge_body,
        out_type=(
            jax.ShapeDtypeStruct((NC, NPAD, D_OUT), jnp.float32),
            jax.ShapeDtypeStruct((NW, NPAD), jnp.float32),
        ),
        mesh=plsc.VectorSubcoreMesh(core_axis_name="c", subcore_axis_name="s"),
        compiler_params=pltpu.CompilerParams(needs_layout_passes=False),
        scratch_types=[
            pltpu.VMEM((S,), jnp.int32),
            pltpu.VMEM((S,), jnp.int32),
            pltpu.VMEM((S,), jnp.int32),
            pltpu.VMEM((S,), jnp.int32),
            pltpu.VMEM((S * T,), jnp.float32),
            pltpu.VMEM((S * T,), jnp.float32),
            pltpu.VMEM((S,), jnp.int32),
            pltpu.VMEM((S,), jnp.int32),
            pltpu.VMEM((S, T * D_OUT // 2), jnp.int32),
            pltpu.VMEM((S, T * D_OUT // 2), jnp.int32),
            pltpu.VMEM((S, D_OUT), jnp.float32),
            pltpu.VMEM((S, D_OUT), jnp.float32),
            pltpu.VMEM((NPAD,), jnp.float32),
            pltpu.VMEM_SHARED((NPAD, D_OUT), jnp.float32),
            pltpu.SemaphoreType.DMA,
            pltpu.SemaphoreType.DMA,
            pltpu.SemaphoreType.DMA,
            pltpu.SemaphoreType.DMA,
            pltpu.SemaphoreType.DMA,
            pltpu.SemaphoreType.DMA,
        ],
    )
    parts, degparts = edge_fn(y3, src_l, tgt_l, a0_l, a1_l, a2_l, a3_l)

    recip = pl.pallas_call(
        _deg_body,
        grid=(NPAD // BD,),
        in_specs=[pl.BlockSpec((NW, BD), lambda i: (0, i))],
        out_specs=pl.BlockSpec((BD, 1), lambda i: (i, 0)),
        out_shape=jax.ShapeDtypeStruct((NPAD, 1), jnp.float32),
    )(degparts)

    out = pl.pallas_call(
        _final_body,
        grid=(N_NODES // BF,),
        in_specs=[
            pl.BlockSpec((NC, BF, D_OUT), lambda i: (0, i, 0)),
            pl.BlockSpec((BF, 1), lambda i: (i, 0)),
            pl.BlockSpec((BF, D_OUT), lambda i: (i, 0)),
            pl.BlockSpec((1, D_OUT), lambda i: (0, 0)),
        ],
        out_specs=pl.BlockSpec((BF, D_OUT), lambda i: (i, 0)),
        out_shape=jax.ShapeDtypeStruct((N_NODES, D_OUT), jnp.float32),
    )(parts, recip, s, bias.reshape(1, D_OUT))
    return out
